# baseline (speedup 1.0000x reference)
import functools

import jax
import jax.numpy as jnp
from jax import lax
from jax.experimental import pallas as pl
from jax.experimental.pallas import tpu as pltpu
from jax.experimental.pallas import tpu_sc as plsc

NC = 2
NS = 16
LB = 128
NPH = 3
D = 128


def _sc_mesh():
    return plsc.VectorSubcoreMesh(core_axis_name="c", subcore_axis_name="s")


def _sc_degree(dstb, ones_blk, zeros_blk, np_, kph):
    rows_per_tile = np_ // NS
    chunks = rows_per_tile // LB

    @functools.partial(
        pl.kernel,
        mesh=_sc_mesh(),
        out_type=jax.ShapeDtypeStruct((NC, np_), jnp.float32),
        scratch_types=[
            pltpu.VMEM((kph, LB), jnp.int32),
            pltpu.VMEM((LB,), jnp.float32),
            pltpu.VMEM((LB,), jnp.float32),
            pltpu.VMEM_SHARED((np_,), jnp.float32),
        ],
    )
    def deg_kernel(dst_h, ones_h, z_h, out_h, dstv, onesv, vbuf, acc):
        cid = lax.axis_index("c")
        sid = lax.axis_index("s")
        base = sid * rows_per_tile
        pltpu.sync_copy(ones_h, onesv)
        pltpu.sync_copy(z_h, vbuf)
        for k in range(chunks):
            pltpu.sync_copy(vbuf, acc.at[pl.ds(base + k * LB, LB)])
        plsc.subcore_barrier()

        for q in range(NPH):
            pltpu.sync_copy(dst_h.at[cid, sid, q], dstv)

            def body(j, carry):
                pltpu.sync_copy(onesv, acc.at[dstv.at[j]], add=True)
                return carry

            lax.fori_loop(0, kph, body, 0)
        plsc.subcore_barrier()
        for k in range(chunks):
            pltpu.sync_copy(acc.at[pl.ds(base + k * LB, LB)], vbuf)
            pltpu.sync_copy(vbuf, out_h.at[cid, pl.ds(base + k * LB, LB)])

    return deg_kernel(dstb, ones_blk, zeros_blk)


def _sc_aggregate(table, srcb, dstb, zeros_blk, np_, kph):
    rows_per_tile = np_ // NS
    chunks = rows_per_tile // LB

    @functools.partial(
        pl.kernel,
        mesh=_sc_mesh(),
        out_type=jax.ShapeDtypeStruct((NC, np_, D), jnp.float32),
        scratch_types=[
            pltpu.VMEM((kph, LB), jnp.int32),
            pltpu.VMEM((kph, LB), jnp.int32),
            pltpu.VMEM((LB, D), jnp.float32),
            pltpu.VMEM((LB, D), jnp.float32),
            pltpu.VMEM_SHARED((np_, D), jnp.float32),
            pltpu.SemaphoreType.DMA,
            pltpu.SemaphoreType.DMA,
        ],
    )
    def agg_kernel(table_h, src_h, dst_h, z_h, out_h,
                   srcv, dstv, gbuf0, gbuf1, acc, sem0, sem1):
        cid = lax.axis_index("c")
        sid = lax.axis_index("s")
        base = sid * rows_per_tile
        pltpu.sync_copy(z_h, gbuf0)
        for k in range(chunks):
            pltpu.sync_copy(gbuf0, acc.at[pl.ds(base + k * LB, LB)])
        plsc.subcore_barrier()

        for q in range(NPH):
            pltpu.sync_copy(src_h.at[cid, sid, q], srcv)
            pltpu.sync_copy(dst_h.at[cid, sid, q], dstv)
            def body(j, carry):
                pltpu.async_copy(table_h.at[srcv.at[j]], gbuf0, sem0).wait()
                pltpu.sync_copy(gbuf0, acc.at[dstv.at[j]], add=True)
                return carry

            lax.fori_loop(0, kph, body, 0)
        plsc.subcore_barrier()
        for k in range(chunks):
            pltpu.sync_copy(acc.at[pl.ds(base + k * LB, LB)], gbuf0)
            pltpu.sync_copy(gbuf0, out_h.at[cid, pl.ds(base + k * LB, LB)])

    return agg_kernel(table, srcb, dstb, zeros_blk)


def _dinv_of(d0, d1):
    deg = d0 + d1
    return jnp.where(deg > 0.0, lax.rsqrt(deg), 0.0)


_BLK = 1024


def _tc_prescale(x, d0, d1, np_):
    def f(x_ref, d0_ref, d1_ref, o_ref):
        o_ref[...] = x_ref[...] * _dinv_of(d0_ref[...], d1_ref[...])

    return pl.pallas_call(
        f,
        grid=(np_ // _BLK,),
        in_specs=[
            pl.BlockSpec((_BLK, D), lambda i: (i, 0)),
            pl.BlockSpec((_BLK, 1), lambda i: (i, 0)),
            pl.BlockSpec((_BLK, 1), lambda i: (i, 0)),
        ],
        out_specs=pl.BlockSpec((_BLK, D), lambda i: (i, 0)),
        out_shape=jax.ShapeDtypeStruct((np_, D), jnp.float32),
    )(x, d0, d1)


def _tc_layer(p0, p1, d0, d1, w, b, np_):
    def f(p0_ref, p1_ref, d0_ref, d1_ref, w_ref, b_ref, o_ref):
        dinv = _dinv_of(d0_ref[...], d1_ref[...])
        agg = (p0_ref[...] + p1_ref[...]) * dinv
        h = jnp.dot(agg, w_ref[...], preferred_element_type=jnp.float32)
        h = jnp.maximum(h + b_ref[...], 0.0)
        o_ref[...] = h * dinv

    return pl.pallas_call(
        f,
        grid=(np_ // _BLK,),
        in_specs=[
            pl.BlockSpec((_BLK, D), lambda i: (i, 0)),
            pl.BlockSpec((_BLK, D), lambda i: (i, 0)),
            pl.BlockSpec((_BLK, 1), lambda i: (i, 0)),
            pl.BlockSpec((_BLK, 1), lambda i: (i, 0)),
            pl.BlockSpec((D, D), lambda i: (0, 0)),
            pl.BlockSpec((1, D), lambda i: (0, 0)),
        ],
        out_specs=pl.BlockSpec((_BLK, D), lambda i: (i, 0)),
        out_shape=jax.ShapeDtypeStruct((np_, D), jnp.float32),
    )(p0, p1, d0, d1, w, b)


def _tc_final(p0, p1, d0, d1, wm, bm, eps, rad, np_, z, max_clamp, tiny):
    def f(p0_ref, p1_ref, d0_ref, d1_ref, wm_ref, bm_ref, eps_ref, rad_ref,
          z0s_ref, mhs_ref, std_ref, vs_ref, us_ref, ex_ref):
        radius = rad_ref[0, 0]
        dinv = _dinv_of(d0_ref[...], d1_ref[...])
        agg = (p0_ref[...] + p1_ref[...]) * dinv
        t = jnp.dot(agg, wm_ref[...], preferred_element_type=jnp.float32)
        t = t + bm_ref[...]
        mu = jnp.clip(t[:, :z], -max_clamp, max_clamp)
        logvar = t[:, z:]
        mn = jnp.sqrt(jnp.sum(mu * mu, axis=1, keepdims=True))
        mn = jnp.maximum(mn, tiny)
        r = mn / radius
        er = jnp.exp(r)
        ch = 0.5 * (er + 1.0 / er)
        sh = 0.5 * (er - 1.0 / er)
        mht = radius * ch
        mhs = (radius * sh / mn) * mu
        std = jnp.maximum(logvar, 0.0) + jnp.log1p(jnp.exp(-jnp.abs(logvar)))
        std = std + 1e-5
        vt = eps_ref[...] * std
        coef = jnp.sum(mhs * vt, axis=1, keepdims=True) / (radius * (radius + mht))
        ut = coef * (radius + mht)
        us = vt + coef * mhs
        lp = jnp.sum(us * us, axis=1, keepdims=True) - ut * ut
        un = jnp.sqrt(jnp.maximum(lp, tiny))
        rr = un / radius
        er2 = jnp.exp(rr)
        ch2 = 0.5 * (er2 + 1.0 / er2)
        sh2 = 0.5 * (er2 - 1.0 / er2)
        s_un = sh2 * radius / un
        z0t = ch2 * mht + s_un * ut
        z0s = ch2 * mhs + s_un * us
        lane = lax.broadcasted_iota(jnp.int32, (t.shape[0], D), 1)
        ex = (jnp.where(lane == 0, mht, 0.0)
              + jnp.where(lane == 1, ut, 0.0)
              + jnp.where(lane == 2, z0t, 0.0))
        z0s_ref[...] = z0s
        mhs_ref[...] = mhs
        std_ref[...] = std
        vs_ref[...] = vt
        us_ref[...] = us
        ex_ref[...] = ex

    row = lambda i: (i, 0)
    full = lambda i: (0, 0)
    sds = jax.ShapeDtypeStruct
    return pl.pallas_call(
        f,
        grid=(np_ // _BLK,),
        in_specs=[
            pl.BlockSpec((_BLK, D), row),
            pl.BlockSpec((_BLK, D), row),
            pl.BlockSpec((_BLK, 1), row),
            pl.BlockSpec((_BLK, 1), row),
            pl.BlockSpec((D, 2 * z), full),
            pl.BlockSpec((1, 2 * z), full),
            pl.BlockSpec((_BLK, z), row),
            pl.BlockSpec((1, 1), full),
        ],
        out_specs=[
            pl.BlockSpec((_BLK, z), row),
            pl.BlockSpec((_BLK, z), row),
            pl.BlockSpec((_BLK, z), row),
            pl.BlockSpec((_BLK, z), row),
            pl.BlockSpec((_BLK, z), row),
            pl.BlockSpec((_BLK, D), row),
        ],
        out_shape=[
            sds((np_, z), jnp.float32),
            sds((np_, z), jnp.float32),
            sds((np_, z), jnp.float32),
            sds((np_, z), jnp.float32),
            sds((np_, z), jnp.float32),
            sds((np_, D), jnp.float32),
        ],
    )(p0, p1, d0, d1, wm, bm, eps, rad)


def kernel(x, edge_index, W1, b1, W2, b2, W_mu, b_mu, W_lv, b_lv, radius):
    n = x.shape[0]
    e = edge_index.shape[1]
    z = W_mu.shape[1]
    np_ = -(-n // (NS * LB)) * (NS * LB)
    et = e + n
    kblk = -(-et // (NC * NS * LB))
    kblk = -(-kblk // (2 * NPH)) * (2 * NPH)
    kph = kblk // NPH
    ep = kblk * NC * NS * LB

    loop = jnp.arange(n, dtype=edge_index.dtype)
    padi = jnp.full((ep - et,), n, edge_index.dtype)
    src = jnp.concatenate([edge_index[0], loop, padi]).reshape(
        NC, NS, NPH, kph, LB)
    dst = jnp.concatenate([edge_index[1], loop, padi]).reshape(
        NC, NS, NPH, kph, LB)

    z2d = jnp.zeros((LB, D), jnp.float32)
    z1d = jnp.zeros((LB,), jnp.float32)
    o1d = jnp.ones((LB,), jnp.float32)
    xp = jnp.pad(x, ((0, np_ - n), (0, 0)))

    degp = _sc_degree(dst, o1d, z1d, np_, kph)
    d0 = degp[0][:, None]
    d1 = degp[1][:, None]

    xs = _tc_prescale(xp, d0, d1, np_)
    p = _sc_aggregate(xs, src, dst, z2d, np_, kph)
    h1s = _tc_layer(p[0], p[1], d0, d1, W1, b1[None, :], np_)
    p = _sc_aggregate(h1s, src, dst, z2d, np_, kph)
    h2s = _tc_layer(p[0], p[1], d0, d1, W2, b2[None, :], np_)
    p = _sc_aggregate(h2s, src, dst, z2d, np_, kph)

    wm = jnp.concatenate([W_mu, W_lv], axis=1)
    bm = jnp.concatenate([b_mu, b_lv])[None, :]
    eps = jax.random.normal(jax.random.key(7), (n, z), dtype=x.dtype)
    epsp = jnp.pad(eps, ((0, np_ - n), (0, 0)))
    rad = radius.reshape(1, 1)

    z0s, mhs, stdf, vs, us, ex = _tc_final(
        p[0], p[1], d0, d1, wm, bm, epsp, rad, np_, z, 40.0, 1e-8)

    z0 = jnp.concatenate([ex[:n, 2:3], z0s[:n]], axis=1)
    mu_h = jnp.concatenate([ex[:n, 0:1], mhs[:n]], axis=1)
    v = jnp.concatenate([jnp.zeros((n, 1), x.dtype), vs[:n]], axis=1)
    u = jnp.concatenate([ex[:n, 1:2], us[:n]], axis=1)
    return z0, mu_h, stdf[:n], v, u


# speedup vs baseline: 4.2251x; 4.2251x over previous
import functools

import jax
import jax.numpy as jnp
from jax import lax
from jax.experimental import pallas as pl
from jax.experimental.pallas import tpu as pltpu
from jax.experimental.pallas import tpu_sc as plsc

NC = 2
NS = 16
LB = 128
NPH = 3
D = 128


def _sc_mesh():
    return plsc.VectorSubcoreMesh(core_axis_name="c", subcore_axis_name="s")


def _sc_degree(dstb, ones_blk, zeros_blk, np_, kph):
    rows_per_tile = np_ // NS
    chunks = rows_per_tile // LB

    @functools.partial(
        pl.kernel,
        mesh=_sc_mesh(),
        out_type=jax.ShapeDtypeStruct((NC, np_), jnp.float32),
        scratch_types=[
            pltpu.VMEM((kph, LB), jnp.int32),
            pltpu.VMEM((LB,), jnp.float32),
            pltpu.VMEM((LB,), jnp.float32),
            pltpu.VMEM_SHARED((np_,), jnp.float32),
        ],
    )
    def deg_kernel(dst_h, ones_h, z_h, out_h, dstv, onesv, vbuf, acc):
        cid = lax.axis_index("c")
        sid = lax.axis_index("s")
        base = sid * rows_per_tile
        pltpu.sync_copy(ones_h, onesv)
        pltpu.sync_copy(z_h, vbuf)
        for k in range(chunks):
            pltpu.sync_copy(vbuf, acc.at[pl.ds(base + k * LB, LB)])
        plsc.subcore_barrier()

        for q in range(NPH):
            pltpu.sync_copy(dst_h.at[cid, sid, q], dstv)

            def body(j, carry):
                pltpu.sync_copy(onesv, acc.at[dstv.at[j]], add=True)
                return carry

            lax.fori_loop(0, kph, body, 0)
        plsc.subcore_barrier()
        for k in range(chunks):
            pltpu.sync_copy(acc.at[pl.ds(base + k * LB, LB)], vbuf)
            pltpu.sync_copy(vbuf, out_h.at[cid, pl.ds(base + k * LB, LB)])

    return deg_kernel(dstb, ones_blk, zeros_blk)


def _sc_aggregate(table, srcb, dstb, zeros_blk, np_, kph):
    rows_per_tile = np_ // NS
    chunks = rows_per_tile // LB

    @functools.partial(
        pl.kernel,
        mesh=_sc_mesh(),
        out_type=jax.ShapeDtypeStruct((NC, np_, D), jnp.float32),
        scratch_types=[
            pltpu.VMEM((kph, LB), jnp.int32),
            pltpu.VMEM((kph, LB), jnp.int32),
            pltpu.VMEM((LB, D), jnp.float32),
            pltpu.VMEM((LB, D), jnp.float32),
            pltpu.VMEM_SHARED((np_, D), jnp.float32),
            pltpu.SemaphoreType.DMA,
            pltpu.SemaphoreType.DMA,
        ],
    )
    def agg_kernel(table_h, src_h, dst_h, z_h, out_h,
                   srcv, dstv, gbuf0, gbuf1, acc, sem0, sem1):
        cid = lax.axis_index("c")
        sid = lax.axis_index("s")
        base = sid * rows_per_tile
        pltpu.sync_copy(z_h, gbuf0)
        for k in range(chunks):
            pltpu.sync_copy(gbuf0, acc.at[pl.ds(base + k * LB, LB)])
        plsc.subcore_barrier()

        for q in range(NPH):
            pltpu.sync_copy(src_h.at[cid, sid, q], srcv)
            pltpu.sync_copy(dst_h.at[cid, sid, q], dstv)
            pltpu.async_copy(table_h.at[srcv.at[0]], gbuf0, sem0)

            def pair(j2, carry):
                j = j2 * 2
                pltpu.make_async_copy(table_h.at[srcv.at[j]],
                                      gbuf0, sem0).wait()
                pltpu.async_copy(table_h.at[srcv.at[j + 1]], gbuf1, sem1)
                pltpu.sync_copy(gbuf0, acc.at[dstv.at[j]], add=True)
                pltpu.make_async_copy(table_h.at[srcv.at[j + 1]],
                                      gbuf1, sem1).wait()

                @pl.when(j2 + 1 < kph // 2)
                def _():
                    pltpu.async_copy(table_h.at[srcv.at[j + 2]],
                                     gbuf0, sem0)
                pltpu.sync_copy(gbuf1, acc.at[dstv.at[j + 1]], add=True)
                return carry

            lax.fori_loop(0, kph // 2, pair, 0)
        plsc.subcore_barrier()
        for k in range(chunks):
            pltpu.sync_copy(acc.at[pl.ds(base + k * LB, LB)], gbuf0)
            pltpu.sync_copy(gbuf0, out_h.at[cid, pl.ds(base + k * LB, LB)])

    return agg_kernel(table, srcb, dstb, zeros_blk)


def _dinv_of(d0, d1):
    deg = d0 + d1
    return jnp.where(deg > 0.0, lax.rsqrt(deg), 0.0)


_BLK = 1024


def _tc_prescale(x, d0, d1, np_):
    def f(x_ref, d0_ref, d1_ref, o_ref):
        o_ref[...] = x_ref[...] * _dinv_of(d0_ref[...], d1_ref[...])

    return pl.pallas_call(
        f,
        grid=(np_ // _BLK,),
        in_specs=[
            pl.BlockSpec((_BLK, D), lambda i: (i, 0)),
            pl.BlockSpec((_BLK, 1), lambda i: (i, 0)),
            pl.BlockSpec((_BLK, 1), lambda i: (i, 0)),
        ],
        out_specs=pl.BlockSpec((_BLK, D), lambda i: (i, 0)),
        out_shape=jax.ShapeDtypeStruct((np_, D), jnp.float32),
    )(x, d0, d1)


def _tc_layer(p0, p1, d0, d1, w, b, np_):
    def f(p0_ref, p1_ref, d0_ref, d1_ref, w_ref, b_ref, o_ref):
        dinv = _dinv_of(d0_ref[...], d1_ref[...])
        agg = (p0_ref[...] + p1_ref[...]) * dinv
        h = jnp.dot(agg, w_ref[...], preferred_element_type=jnp.float32)
        h = jnp.maximum(h + b_ref[...], 0.0)
        o_ref[...] = h * dinv

    return pl.pallas_call(
        f,
        grid=(np_ // _BLK,),
        in_specs=[
            pl.BlockSpec((_BLK, D), lambda i: (i, 0)),
            pl.BlockSpec((_BLK, D), lambda i: (i, 0)),
            pl.BlockSpec((_BLK, 1), lambda i: (i, 0)),
            pl.BlockSpec((_BLK, 1), lambda i: (i, 0)),
            pl.BlockSpec((D, D), lambda i: (0, 0)),
            pl.BlockSpec((1, D), lambda i: (0, 0)),
        ],
        out_specs=pl.BlockSpec((_BLK, D), lambda i: (i, 0)),
        out_shape=jax.ShapeDtypeStruct((np_, D), jnp.float32),
    )(p0, p1, d0, d1, w, b)


def _tc_final(p0, p1, d0, d1, wm, bm, eps, rad, np_, z, max_clamp, tiny):
    def f(p0_ref, p1_ref, d0_ref, d1_ref, wm_ref, bm_ref, eps_ref, rad_ref,
          z0s_ref, mhs_ref, std_ref, vs_ref, us_ref, ex_ref):
        radius = rad_ref[0, 0]
        dinv = _dinv_of(d0_ref[...], d1_ref[...])
        agg = (p0_ref[...] + p1_ref[...]) * dinv
        t = jnp.dot(agg, wm_ref[...], preferred_element_type=jnp.float32)
        t = t + bm_ref[...]
        mu = jnp.clip(t[:, :z], -max_clamp, max_clamp)
        logvar = t[:, z:]
        mn = jnp.sqrt(jnp.sum(mu * mu, axis=1, keepdims=True))
        mn = jnp.maximum(mn, tiny)
        r = mn / radius
        er = jnp.exp(r)
        ch = 0.5 * (er + 1.0 / er)
        sh = 0.5 * (er - 1.0 / er)
        mht = radius * ch
        mhs = (radius * sh / mn) * mu
        std = jnp.maximum(logvar, 0.0) + jnp.log1p(jnp.exp(-jnp.abs(logvar)))
        std = std + 1e-5
        vt = eps_ref[...] * std
        coef = jnp.sum(mhs * vt, axis=1, keepdims=True) / (radius * (radius + mht))
        ut = coef * (radius + mht)
        us = vt + coef * mhs
        lp = jnp.sum(us * us, axis=1, keepdims=True) - ut * ut
        un = jnp.sqrt(jnp.maximum(lp, tiny))
        rr = un / radius
        er2 = jnp.exp(rr)
        ch2 = 0.5 * (er2 + 1.0 / er2)
        sh2 = 0.5 * (er2 - 1.0 / er2)
        s_un = sh2 * radius / un
        z0t = ch2 * mht + s_un * ut
        z0s = ch2 * mhs + s_un * us
        lane = lax.broadcasted_iota(jnp.int32, (t.shape[0], D), 1)
        ex = (jnp.where(lane == 0, mht, 0.0)
              + jnp.where(lane == 1, ut, 0.0)
              + jnp.where(lane == 2, z0t, 0.0))
        z0s_ref[...] = z0s
        mhs_ref[...] = mhs
        std_ref[...] = std
        vs_ref[...] = vt
        us_ref[...] = us
        ex_ref[...] = ex

    row = lambda i: (i, 0)
    full = lambda i: (0, 0)
    sds = jax.ShapeDtypeStruct
    return pl.pallas_call(
        f,
        grid=(np_ // _BLK,),
        in_specs=[
            pl.BlockSpec((_BLK, D), row),
            pl.BlockSpec((_BLK, D), row),
            pl.BlockSpec((_BLK, 1), row),
            pl.BlockSpec((_BLK, 1), row),
            pl.BlockSpec((D, 2 * z), full),
            pl.BlockSpec((1, 2 * z), full),
            pl.BlockSpec((_BLK, z), row),
            pl.BlockSpec((1, 1), full),
        ],
        out_specs=[
            pl.BlockSpec((_BLK, z), row),
            pl.BlockSpec((_BLK, z), row),
            pl.BlockSpec((_BLK, z), row),
            pl.BlockSpec((_BLK, z), row),
            pl.BlockSpec((_BLK, z), row),
            pl.BlockSpec((_BLK, D), row),
        ],
        out_shape=[
            sds((np_, z), jnp.float32),
            sds((np_, z), jnp.float32),
            sds((np_, z), jnp.float32),
            sds((np_, z), jnp.float32),
            sds((np_, z), jnp.float32),
            sds((np_, D), jnp.float32),
        ],
    )(p0, p1, d0, d1, wm, bm, eps, rad)


def kernel(x, edge_index, W1, b1, W2, b2, W_mu, b_mu, W_lv, b_lv, radius):
    n = x.shape[0]
    e = edge_index.shape[1]
    z = W_mu.shape[1]
    np_ = -(-n // (NS * LB)) * (NS * LB)
    et = e + n
    kblk = -(-et // (NC * NS * LB))
    kblk = -(-kblk // (2 * NPH)) * (2 * NPH)
    kph = kblk // NPH
    ep = kblk * NC * NS * LB

    loop = jnp.arange(n, dtype=edge_index.dtype)
    padi = (n + jnp.arange(ep - et, dtype=edge_index.dtype) % (np_ - n))
    src = jnp.concatenate([edge_index[0], loop, padi]).reshape(
        NC, NS, NPH, kph, LB)
    dst = jnp.concatenate([edge_index[1], loop, padi]).reshape(
        NC, NS, NPH, kph, LB)

    z2d = jnp.zeros((LB, D), jnp.float32)
    z1d = jnp.zeros((LB,), jnp.float32)
    o1d = jnp.ones((LB,), jnp.float32)
    xp = jnp.pad(x, ((0, np_ - n), (0, 0)))

    degp = _sc_degree(dst, o1d, z1d, np_, kph)
    d0 = degp[0][:, None]
    d1 = degp[1][:, None]

    xs = _tc_prescale(xp, d0, d1, np_)
    p = _sc_aggregate(xs, src, dst, z2d, np_, kph)
    h1s = _tc_layer(p[0], p[1], d0, d1, W1, b1[None, :], np_)
    p = _sc_aggregate(h1s, src, dst, z2d, np_, kph)
    h2s = _tc_layer(p[0], p[1], d0, d1, W2, b2[None, :], np_)
    p = _sc_aggregate(h2s, src, dst, z2d, np_, kph)

    wm = jnp.concatenate([W_mu, W_lv], axis=1)
    bm = jnp.concatenate([b_mu, b_lv])[None, :]
    eps = jax.random.normal(jax.random.key(7), (n, z), dtype=x.dtype)
    epsp = jnp.pad(eps, ((0, np_ - n), (0, 0)))
    rad = radius.reshape(1, 1)

    z0s, mhs, stdf, vs, us, ex = _tc_final(
        p[0], p[1], d0, d1, wm, bm, epsp, rad, np_, z, 40.0, 1e-8)

    z0 = jnp.concatenate([ex[:n, 2:3], z0s[:n]], axis=1)
    mu_h = jnp.concatenate([ex[:n, 0:1], mhs[:n]], axis=1)
    v = jnp.concatenate([jnp.zeros((n, 1), x.dtype), vs[:n]], axis=1)
    u = jnp.concatenate([ex[:n, 1:2], us[:n]], axis=1)
    return z0, mu_h, stdf[:n], v, u
